# baseline (speedup 1.0000x reference)
import functools
import jax
import jax.numpy as jnp
from jax import lax
from jax.experimental import pallas as pl
from jax.experimental.pallas import tpu as pltpu
from jax.experimental.pallas import tpu_sc as plsc

_LS = 0.1
_N = 32000
_B = 2048
_CONF = 1.0 - _LS
_SV = _LS / (_N - 1)


_NW = 32
_RPW = _B // _NW
_CH = 128
_CH2 = 256


def _sc_scan_kernel(t_hbm, x_hbm, val_hbm, tbuf, xbuf, buf2, idxbuf, fvbuf,
                    gbuf, valbuf, sem, gsem):
    wid = lax.axis_index("s") * 2 + lax.axis_index("c")
    base = wid * _RPW
    iota = lax.iota(jnp.int32, 16)

    xcopy = pltpu.async_copy(x_hbm.at[pl.ds(base, _RPW), pl.ds(0, _CH)],
                             xbuf, gsem)
    pltpu.sync_copy(t_hbm.at[pl.ds(base, _RPW), pl.ds(0, _CH)], tbuf)

    def scan_row(r):
        fvbuf[...] = jnp.full((16,), -1, jnp.int32)

        def body1(j, carry):
            v = tbuf[r, pl.ds(j * 16, 16)]
            ffs = plsc.all_reduce_ffs(v > 0)
            cand = jnp.where(ffs < 16, ffs + j * 16, -1)
            old = fvbuf[...]
            fvbuf[...] = jnp.where(old >= 0, old, cand)
            return carry

        lax.fori_loop(0, _CH // 16, body1, jnp.int32(0))

        row_al = base + (r & ~7)

        @pl.when(fvbuf[...][0] < 0)
        def _continue():
            def chunk_step(k, carry):
                @pl.when(fvbuf[...][0] < 0)
                def _do():
                    off = pl.multiple_of(k * _CH2, 128)
                    pltpu.sync_copy(
                        t_hbm.at[pl.ds(row_al, 8), pl.ds(off, _CH2)], buf2)

                    def inner(j, c2):
                        v = buf2[r & 7, pl.ds(j * 16, 16)]
                        ffs = plsc.all_reduce_ffs(v > 0)
                        cand = jnp.where(ffs < 16,
                                         k * _CH2 + j * 16 + ffs, -1)
                        old = fvbuf[...]
                        fvbuf[...] = jnp.where(old >= 0, old, cand)
                        return c2

                    lax.fori_loop(0, _CH2 // 16, inner, jnp.int32(0))

                return carry

            lax.fori_loop(0, _N // _CH2, chunk_step, jnp.int32(0))

        fv = fvbuf[...]
        return jnp.where(fv < 0, 0, fv)

    for g in range(_RPW // 16):
        vec = jnp.zeros((16,), jnp.int32)
        for r in range(16):
            idx = scan_row(g * 16 + r)
            vec = jnp.where(iota == r, idx, vec)
        idxbuf[pl.ds(g * 16, 16)] = vec

    xcopy.wait()
    for g in range(_RPW // 16):
        grp = idxbuf[pl.ds(g * 16, 16)]
        cl = jnp.where(grp < _CH, grp, 0)
        vals = plsc.load_gather(xbuf, [iota + g * 16, cl])
        valbuf[pl.ds(g * 16, 16)] = vals

        far = plsc.all_reduce_ffs(grp >= _CH)

        @pl.when(far[0] < 16)
        def _rare(g=g, grp=grp, vals=vals):
            copies = []
            for r in range(16):
                col_al = pl.multiple_of((grp[r] // 128) * 128, 128)
                row_al = pl.multiple_of(base + ((g * 16 + r) & ~7), 8)
                copies.append(pltpu.async_copy(
                    x_hbm.at[pl.ds(row_al, 8), pl.ds(col_al, 128)],
                    gbuf.at[r], gsem))
            for c in copies:
                c.wait()
            tvals = plsc.load_gather(gbuf, [iota, iota & 7, grp & 127])
            valbuf[pl.ds(g * 16, 16)] = jnp.where(grp < _CH, vals, tvals)

    pltpu.sync_copy(valbuf, val_hbm.at[pl.ds(base, _RPW)])


@functools.partial(
    pl.kernel,
    mesh=plsc.VectorSubcoreMesh(core_axis_name="c", subcore_axis_name="s"),
    out_type=jax.ShapeDtypeStruct((_B,), jnp.float32),
    scratch_types=[
        pltpu.VMEM((_RPW, _CH), jnp.int32),
        pltpu.VMEM((_RPW, _CH), jnp.float32),
        pltpu.VMEM((8, _CH2), jnp.int32),
        pltpu.VMEM((_RPW,), jnp.int32),
        pltpu.VMEM((16,), jnp.int32),
        pltpu.VMEM((16, 8, 128), jnp.float32),
        pltpu.VMEM((_RPW,), jnp.float32),
        pltpu.SemaphoreType.DMA,
        pltpu.SemaphoreType.DMA,
    ],
    compiler_params=pltpu.CompilerParams(needs_layout_passes=False),
)
def _sc_first_one_val(t_hbm, x_hbm, val_hbm, tbuf, xbuf, buf2, idxbuf, fvbuf,
                      gbuf, valbuf, sem, gsem):
    _sc_scan_kernel(t_hbm, x_hbm, val_hbm, tbuf, xbuf, buf2, idxbuf, fvbuf,
                    gbuf, valbuf, sem, gsem)


_RB = 1024
_CB = 3200
_NRB = _B // _RB
_NCB = _N // _CB


def _stats_kernel(x_ref, lse_ref, rsum_ref, m_ref, s_ref, rs_ref):
    ci = pl.program_id(1)
    x = x_ref[...]

    bm = jnp.max(x, axis=1, keepdims=True)
    bsum = jnp.sum(x, axis=1, keepdims=True)

    @pl.when(ci == 0)
    def _init():
        m_ref[...] = bm
        s_ref[...] = jnp.sum(jnp.exp(x - bm), axis=1, keepdims=True)
        rs_ref[...] = bsum

    @pl.when(ci != 0)
    def _update():
        m_old = m_ref[...]
        new_m = jnp.maximum(m_old, bm)
        s_ref[...] = s_ref[...] * jnp.exp(m_old - new_m) + jnp.sum(
            jnp.exp(x - new_m), axis=1, keepdims=True)
        m_ref[...] = new_m
        rs_ref[...] = rs_ref[...] + bsum

    @pl.when(ci == _NCB - 1)
    def _finalize():
        lse_ref[...] = m_ref[...] + jnp.log(s_ref[...])
        rsum_ref[...] = rs_ref[...]


def _combine_kernel(lse_ref, rsum_ref, val_ref, out_ref):
    lse = lse_ref[...]
    rs = rsum_ref[...]
    val = val_ref[...].reshape(_B, 1)
    row_loss = _SV * (rs - _N * lse) + (_CONF - _SV) * (val - lse)
    out_ref[...] = -jnp.sum(row_loss, axis=0, keepdims=True) / _B


def kernel(output, target):
    val = _sc_first_one_val(target, output)
    lse, rsum = pl.pallas_call(
        _stats_kernel,
        grid=(_NRB, _NCB),
        in_specs=[
            pl.BlockSpec((_RB, _CB), lambda ri, ci: (ri, ci)),
        ],
        out_specs=[
            pl.BlockSpec((_RB, 1), lambda ri, ci: (ri, 0)),
            pl.BlockSpec((_RB, 1), lambda ri, ci: (ri, 0)),
        ],
        out_shape=[
            jax.ShapeDtypeStruct((_B, 1), jnp.float32),
            jax.ShapeDtypeStruct((_B, 1), jnp.float32),
        ],
        scratch_shapes=[
            pltpu.VMEM((_RB, 1), jnp.float32),
            pltpu.VMEM((_RB, 1), jnp.float32),
            pltpu.VMEM((_RB, 1), jnp.float32),
        ],
        compiler_params=pltpu.CompilerParams(
            dimension_semantics=("arbitrary", "arbitrary"),
        ),
    )(output)
    res = pl.pallas_call(
        _combine_kernel,
        out_shape=jax.ShapeDtypeStruct((1, 1), jnp.float32),
    )(lse, rsum, val)
    return res[0, 0]


# speedup vs baseline: 1.0145x; 1.0145x over previous
import functools
import jax
import jax.numpy as jnp
from jax import lax
from jax.experimental import pallas as pl
from jax.experimental.pallas import tpu as pltpu
from jax.experimental.pallas import tpu_sc as plsc

_LS = 0.1
_N = 32000
_B = 2048
_CONF = 1.0 - _LS
_SV = _LS / (_N - 1)


_NW = 32
_RPW = _B // _NW
_CH = 128
_CH2 = 256


def _sc_scan_kernel(t_hbm, x_hbm, val_hbm, tbuf, xbuf, buf2, idxbuf, fvbuf,
                    gbuf, valbuf, sem, gsem):
    wid = lax.axis_index("s") * 2 + lax.axis_index("c")
    base = wid * _RPW
    iota = lax.iota(jnp.int32, 16)

    xcopy = pltpu.async_copy(x_hbm.at[pl.ds(base, _RPW), pl.ds(0, _CH)],
                             xbuf, gsem)
    pltpu.sync_copy(t_hbm.at[pl.ds(base, _RPW), pl.ds(0, _CH)], tbuf)

    def scan_row(r):
        fvbuf[...] = jnp.full((16,), -1, jnp.int32)

        def body1(j, carry):
            v = tbuf[r, pl.ds(j * 16, 16)]
            ffs = plsc.all_reduce_ffs(v > 0)
            cand = jnp.where(ffs < 16, ffs + j * 16, -1)
            old = fvbuf[...]
            fvbuf[...] = jnp.where(old >= 0, old, cand)
            return carry

        lax.fori_loop(0, _CH // 16, body1, jnp.int32(0))

        row_al = base + (r & ~7)

        @pl.when(fvbuf[...][0] < 0)
        def _continue():
            def chunk_step(k, carry):
                @pl.when(fvbuf[...][0] < 0)
                def _do():
                    off = pl.multiple_of(k * _CH2, 128)
                    pltpu.sync_copy(
                        t_hbm.at[pl.ds(row_al, 8), pl.ds(off, _CH2)], buf2)

                    def inner(j, c2):
                        v = buf2[r & 7, pl.ds(j * 16, 16)]
                        ffs = plsc.all_reduce_ffs(v > 0)
                        cand = jnp.where(ffs < 16,
                                         k * _CH2 + j * 16 + ffs, -1)
                        old = fvbuf[...]
                        fvbuf[...] = jnp.where(old >= 0, old, cand)
                        return c2

                    lax.fori_loop(0, _CH2 // 16, inner, jnp.int32(0))

                return carry

            lax.fori_loop(0, _N // _CH2, chunk_step, jnp.int32(0))

        fv = fvbuf[...]
        return jnp.where(fv < 0, 0, fv)

    for g in range(_RPW // 16):
        vec = jnp.zeros((16,), jnp.int32)
        for r in range(16):
            idx = scan_row(g * 16 + r)
            vec = jnp.where(iota == r, idx, vec)
        idxbuf[pl.ds(g * 16, 16)] = vec

    xcopy.wait()
    for g in range(_RPW // 16):
        grp = idxbuf[pl.ds(g * 16, 16)]
        cl = jnp.where(grp < _CH, grp, 0)
        vals = plsc.load_gather(xbuf, [iota + g * 16, cl])
        valbuf[pl.ds(g * 16, 16)] = vals

        far = plsc.all_reduce_ffs(grp >= _CH)

        @pl.when(far[0] < 16)
        def _rare(g=g, grp=grp, vals=vals):
            copies = []
            for r in range(16):
                col_al = pl.multiple_of((grp[r] // 128) * 128, 128)
                row_al = pl.multiple_of(base + ((g * 16 + r) & ~7), 8)
                copies.append(pltpu.async_copy(
                    x_hbm.at[pl.ds(row_al, 8), pl.ds(col_al, 128)],
                    gbuf.at[r], gsem))
            for c in copies:
                c.wait()
            tvals = plsc.load_gather(gbuf, [iota, iota & 7, grp & 127])
            valbuf[pl.ds(g * 16, 16)] = jnp.where(grp < _CH, vals, tvals)

    pltpu.sync_copy(valbuf, val_hbm.at[pl.ds(base, _RPW)])


@functools.partial(
    pl.kernel,
    mesh=plsc.VectorSubcoreMesh(core_axis_name="c", subcore_axis_name="s"),
    out_type=jax.ShapeDtypeStruct((_B,), jnp.float32),
    scratch_types=[
        pltpu.VMEM((_RPW, _CH), jnp.int32),
        pltpu.VMEM((_RPW, _CH), jnp.float32),
        pltpu.VMEM((8, _CH2), jnp.int32),
        pltpu.VMEM((_RPW,), jnp.int32),
        pltpu.VMEM((16,), jnp.int32),
        pltpu.VMEM((16, 8, 128), jnp.float32),
        pltpu.VMEM((_RPW,), jnp.float32),
        pltpu.SemaphoreType.DMA,
        pltpu.SemaphoreType.DMA,
    ],
    compiler_params=pltpu.CompilerParams(needs_layout_passes=False),
)
def _sc_first_one_val(t_hbm, x_hbm, val_hbm, tbuf, xbuf, buf2, idxbuf, fvbuf,
                      gbuf, valbuf, sem, gsem):
    _sc_scan_kernel(t_hbm, x_hbm, val_hbm, tbuf, xbuf, buf2, idxbuf, fvbuf,
                    gbuf, valbuf, sem, gsem)


_RB = 512
_CB = 6400
_NRB = _B // _RB
_NCB = _N // _CB


def _stats_kernel(x_ref, lse_ref, rsum_ref, m_ref, s_ref, rs_ref):
    ci = pl.program_id(1)
    x = x_ref[...]

    bm = jnp.max(x, axis=1, keepdims=True)
    bsum = jnp.sum(x, axis=1, keepdims=True)

    @pl.when(ci == 0)
    def _init():
        m_ref[...] = bm
        s_ref[...] = jnp.sum(jnp.exp(x - bm), axis=1, keepdims=True)
        rs_ref[...] = bsum

    @pl.when(ci != 0)
    def _update():
        m_old = m_ref[...]
        new_m = jnp.maximum(m_old, bm)
        s_ref[...] = s_ref[...] * jnp.exp(m_old - new_m) + jnp.sum(
            jnp.exp(x - new_m), axis=1, keepdims=True)
        m_ref[...] = new_m
        rs_ref[...] = rs_ref[...] + bsum

    @pl.when(ci == _NCB - 1)
    def _finalize():
        lse_ref[...] = m_ref[...] + jnp.log(s_ref[...])
        rsum_ref[...] = rs_ref[...]


def _combine_kernel(lse_ref, rsum_ref, val_ref, out_ref):
    lse = lse_ref[...]
    rs = rsum_ref[...]
    val = val_ref[...].reshape(_B, 1)
    row_loss = _SV * (rs - _N * lse) + (_CONF - _SV) * (val - lse)
    out_ref[...] = -jnp.sum(row_loss, axis=0, keepdims=True) / _B


def kernel(output, target):
    val = _sc_first_one_val(target, output)
    lse, rsum = pl.pallas_call(
        _stats_kernel,
        grid=(_NRB, _NCB),
        in_specs=[
            pl.BlockSpec((_RB, _CB), lambda ri, ci: (ri, ci)),
        ],
        out_specs=[
            pl.BlockSpec((_RB, 1), lambda ri, ci: (ri, 0)),
            pl.BlockSpec((_RB, 1), lambda ri, ci: (ri, 0)),
        ],
        out_shape=[
            jax.ShapeDtypeStruct((_B, 1), jnp.float32),
            jax.ShapeDtypeStruct((_B, 1), jnp.float32),
        ],
        scratch_shapes=[
            pltpu.VMEM((_RB, 1), jnp.float32),
            pltpu.VMEM((_RB, 1), jnp.float32),
            pltpu.VMEM((_RB, 1), jnp.float32),
        ],
        compiler_params=pltpu.CompilerParams(
            dimension_semantics=("arbitrary", "arbitrary"),
        ),
    )(output)
    res = pl.pallas_call(
        _combine_kernel,
        out_shape=jax.ShapeDtypeStruct((1, 1), jnp.float32),
    )(lse, rsum, val)
    return res[0, 0]
